# baseline (speedup 1.0000x reference)
import functools

import jax
import jax.numpy as jnp
from jax.experimental import pallas as pl
from jax.experimental.pallas import tpu as pltpu


def _spatial_body(x_ref, conf_ref, wenc_ref, bvec_ref, w1_ref, w2_ref,
                  hout_ref, pooled_ref, *, H, W, P):
    HW = H * W
    L = P * HW
    hid = wenc_ref.shape[0]

    x = x_ref[0]
    conf = conf_ref[0]

    h = jnp.dot(wenc_ref[...].astype(jnp.bfloat16), x,
                preferred_element_type=jnp.float32)
    h = jnp.maximum(h + bvec_ref[0], 0.0) * conf

    p = jax.lax.broadcasted_iota(jnp.int32, (1, L), 1)
    xcol = p % W
    yrow = (p // W) % H
    m_xm = xcol >= 1
    m_xp = xcol <= W - 2
    m_ym = yrow >= 1
    m_yp = yrow <= H - 2

    def conv3x3(v, w_ref, bias):
        zero = jnp.zeros((), jnp.bfloat16)
        vxm = jnp.where(m_xm, pltpu.roll(v, 1, axis=1), zero)
        vxp = jnp.where(m_xp, pltpu.roll(v, L - 1, axis=1), zero)
        c3 = jnp.concatenate([vxm, v, vxp], axis=0)

        def wgroup(g):
            return jnp.concatenate([w_ref[3 * g], w_ref[3 * g + 1],
                                    w_ref[3 * g + 2]], axis=1).astype(jnp.bfloat16)

        g_m1 = jnp.dot(wgroup(0), c3, preferred_element_type=jnp.float32)
        g_0 = jnp.dot(wgroup(1), c3, preferred_element_type=jnp.float32)
        g_p1 = jnp.dot(wgroup(2), c3, preferred_element_type=jnp.float32)
        zf = jnp.zeros((), jnp.float32)
        g = g_0 + jnp.where(m_ym, pltpu.roll(g_m1, W, axis=1), zf)
        g = g + jnp.where(m_yp, pltpu.roll(g_p1, L - W, axis=1), zf)
        return jnp.maximum(g + bias, 0.0)

    h1 = conv3x3(h.astype(jnp.bfloat16), w1_ref, bvec_ref[1])
    h2 = conv3x3(h1.astype(jnp.bfloat16), w2_ref, bvec_ref[2])

    hout_ref[...] = h2.astype(jnp.bfloat16).reshape(1, hid, L)

    sel = (jax.lax.broadcasted_iota(jnp.int32, (P, L), 1) // HW ==
           jax.lax.broadcasted_iota(jnp.int32, (P, L), 0))
    selw = jnp.where(sel, 1.0 / HW, 0.0)
    pooled = jax.lax.dot_general(selw, h2, (((1,), (1,)), ((), ())),
                                 preferred_element_type=jnp.float32)
    pooled_ref[...] = pooled.reshape(P, 1, hid)


def _run_spatial(x, conf, wenc_bf, bvec, w1_cat, w2_cat, H, W, P):
    M, C, L = x.shape
    N = M * P
    hid = wenc_bf.shape[0]
    body = functools.partial(_spatial_body, H=H, W=W, P=P)
    return pl.pallas_call(
        body,
        out_shape=(jax.ShapeDtypeStruct((M, hid, L), jnp.bfloat16),
                   jax.ShapeDtypeStruct((N, 1, hid), jnp.float32)),
        grid=(M,),
        in_specs=[
            pl.BlockSpec((1, C, L), lambda n: (n, 0, 0)),
            pl.BlockSpec((1, 1, L), lambda n: (n, 0, 0)),
            pl.BlockSpec((hid, C), lambda n: (0, 0)),
            pl.BlockSpec((3, hid, 1), lambda n: (0, 0, 0)),
            pl.BlockSpec((9, hid, hid), lambda n: (0, 0, 0)),
            pl.BlockSpec((9, hid, hid), lambda n: (0, 0, 0)),
        ],
        out_specs=(
            pl.BlockSpec((1, hid, L), lambda n: (n, 0, 0)),
            pl.BlockSpec((P, 1, hid), lambda n: (n, 0, 0)),
        ),
        compiler_params=pltpu.CompilerParams(dimension_semantics=("parallel",)),
    )(x, conf, wenc_bf, bvec, w1_cat, w2_cat)


def _gat_body(pooled_ref, ei_ref, wgat_ref, usrc_ref, udst_ref, bgat_ref,
              wout_ref, c_ref, *, num_layers, heads, P):
    N = pooled_ref.shape[0]
    hid = bgat_ref.shape[2]
    C = wout_ref.shape[0]
    E = ei_ref.shape[1]
    neg_slope = 0.2

    ii = jax.lax.broadcasted_iota(jnp.int32, (N, E), 0)
    don = (ii == ei_ref[1:2, :]).astype(jnp.float32)
    son = (ii == ei_ref[0:1, :]).astype(jnp.float32)
    cnt = jax.lax.dot_general(don, son, (((1,), (1,)), ((), ())),
                              preferred_element_type=jnp.float32)
    ri = jax.lax.broadcasted_iota(jnp.int32, (N, N), 0)
    ci = jax.lax.broadcasted_iota(jnp.int32, (N, N), 1)
    adj = jnp.logical_or(cnt > 0, ri == ci)

    xg = pooled_ref[...].reshape(N, hid)

    for l in range(num_layers):
        h_all = jnp.dot(xg, wgat_ref[l], preferred_element_type=jnp.float32)
        s_all = jax.lax.dot_general(usrc_ref[l], xg, (((0,), (1,)), ((), ())),
                                    preferred_element_type=jnp.float32)
        d_all = jnp.dot(xg, udst_ref[l], preferred_element_type=jnp.float32)
        acc = jnp.zeros((N, hid), jnp.float32)
        for hd in range(heads):
            e = d_all[:, hd:hd + 1] + s_all[hd:hd + 1, :]
            e = jnp.where(e > 0, e, neg_slope * e)
            e = jnp.where(adj, e, -1e9)
            e = e - jnp.max(e, axis=-1, keepdims=True)
            pr = jnp.exp(e)
            pr = pr / jnp.sum(pr, axis=-1, keepdims=True)
            acc = acc + jnp.dot(pr, h_all[:, hd * hid:(hd + 1) * hid],
                                preferred_element_type=jnp.float32)
        xg = jnp.maximum(acc * (1.0 / heads) + bgat_ref[l], 0.0)

    zt = jax.lax.dot_general(xg, wout_ref[...], (((1,), (1,)), ((), ())),
                             preferred_element_type=jnp.float32)
    c_ref[...] = zt.reshape(N // P, P, C)


def _run_gat(pooled, edge_index, w_gat, u_src, u_dst, b_gat, w_out,
             num_layers, heads, P):
    N = pooled.shape[0]
    hid = pooled.shape[2]
    C = w_out.shape[0]
    E = edge_index.shape[1]
    body = functools.partial(_gat_body, num_layers=num_layers, heads=heads, P=P)
    return pl.pallas_call(
        body,
        out_shape=jax.ShapeDtypeStruct((N // P, P, C), jnp.float32),
        grid=(1,),
        in_specs=[
            pl.BlockSpec((N, 1, hid), lambda i: (0, 0, 0)),
            pl.BlockSpec((2, E), lambda i: (0, 0)),
            pl.BlockSpec((num_layers, hid, heads * hid), lambda i: (0, 0, 0)),
            pl.BlockSpec((num_layers, hid, heads), lambda i: (0, 0, 0)),
            pl.BlockSpec((num_layers, hid, heads), lambda i: (0, 0, 0)),
            pl.BlockSpec((num_layers, 1, hid), lambda i: (0, 0, 0)),
            pl.BlockSpec((C, hid), lambda i: (0, 0)),
        ],
        out_specs=pl.BlockSpec((N // P, P, C), lambda i: (0, 0, 0)),
        compiler_params=pltpu.CompilerParams(dimension_semantics=("arbitrary",)),
    )(pooled, edge_index, w_gat, u_src, u_dst, b_gat, w_out)


def _combine_body(h_ref, c_ref, wout_ref, bout_ref, out_ref, *, HW, P, SB):
    C = wout_ref.shape[0]
    L = h_ref.shape[2]
    sel = (jax.lax.broadcasted_iota(jnp.int32, (P, L), 1) // HW ==
           jax.lax.broadcasted_iota(jnp.int32, (P, L), 0))
    selw = jnp.where(sel, 1.0, 0.0)
    wout_bf = wout_ref[...].astype(jnp.bfloat16)
    for k in range(SB):
        y = jnp.dot(wout_bf, h_ref[k], preferred_element_type=jnp.float32)
        cn = c_ref[k]
        corr = jax.lax.dot_general(cn, selw, (((0,), (0,)), ((), ())),
                                   preferred_element_type=jnp.float32)
        out_ref[k] = y + corr + bout_ref[...]


def _run_combine(hproc, cvec, wout_bf, b_out, HW, P, SB):
    M, hid, L = hproc.shape
    C = wout_bf.shape[0]
    body = functools.partial(_combine_body, HW=HW, P=P, SB=SB)
    return pl.pallas_call(
        body,
        out_shape=jax.ShapeDtypeStruct((M, C, L), jnp.float32),
        grid=(M // SB,),
        in_specs=[
            pl.BlockSpec((SB, hid, L), lambda n: (n, 0, 0)),
            pl.BlockSpec((SB, P, C), lambda n: (n, 0, 0)),
            pl.BlockSpec((C, hid), lambda n: (0, 0)),
            pl.BlockSpec((C, 1), lambda n: (0, 0)),
        ],
        out_specs=pl.BlockSpec((SB, C, L), lambda n: (n, 0, 0)),
        compiler_params=pltpu.CompilerParams(dimension_semantics=("parallel",)),
    )(hproc, cvec, wout_bf, b_out)


def kernel(x, edge_index, confidence_maps, w_enc, bvec, w_sp1, w_sp2,
           w_gat, u_src, u_dst, b_gat, w_out, b_out):
    N, C, H, W = x.shape
    HW = H * W
    hid = w_enc.shape[0]
    num_layers = w_gat.shape[0]
    heads = u_src.shape[2]
    P = 4 if N % 4 == 0 else 1
    M = N // P
    SB = 2 if M % 2 == 0 else 1

    x_flat = jnp.transpose(x.reshape(M, P, C, HW), (0, 2, 1, 3)) \
        .reshape(M, C, P * HW).astype(jnp.bfloat16)
    conf_flat = jnp.transpose(confidence_maps.reshape(M, P, 1, HW),
                              (0, 2, 1, 3)).reshape(M, 1, P * HW)

    hproc, pooled = _run_spatial(x_flat, conf_flat, w_enc, bvec,
                                 w_sp1, w_sp2, H, W, P)
    cvec = _run_gat(pooled, edge_index, w_gat, u_src, u_dst, b_gat, w_out,
                    num_layers, heads, P)
    out = _run_combine(hproc, cvec, w_out, b_out, HW, P, SB)
    return jnp.transpose(out.reshape(M, C, P, H, W), (0, 2, 1, 3, 4)) \
        .reshape(N, C, H, W)


# speedup vs baseline: 1.0404x; 1.0404x over previous
import functools

import jax
import jax.numpy as jnp
from jax.experimental import pallas as pl
from jax.experimental.pallas import tpu as pltpu


def _mega_body(x_ref, conf_ref, ei_ref, wenc_ref, bvec_ref, w1_ref, w2_ref,
               wgat_ref, usrc_ref, udst_ref, bgat_ref, wout_ref, bout_ref,
               out_ref, h_sc, pooled_sc, c_sc,
               *, H, W, P, SB, M, num_layers, heads):
    HW = H * W
    L = P * HW
    hid = wenc_ref.shape[0]
    C = wout_ref.shape[0]
    N = M * P
    i = pl.program_id(0)

    @pl.when(i < M)
    def _spatial():
        x = x_ref[0]
        conf = conf_ref[0]

        h = jnp.dot(wenc_ref[...].astype(jnp.bfloat16), x,
                    preferred_element_type=jnp.float32)
        h = jnp.maximum(h + bvec_ref[0], 0.0) * conf

        p = jax.lax.broadcasted_iota(jnp.int32, (1, L), 1)
        xcol = p % W
        yrow = (p // W) % H
        m_xm = xcol >= 1
        m_xp = xcol <= W - 2
        m_ym = yrow >= 1
        m_yp = yrow <= H - 2

        def conv3x3(v, w_ref, bias):
            zero = jnp.zeros((), jnp.bfloat16)
            vxm = jnp.where(m_xm, pltpu.roll(v, 1, axis=1), zero)
            vxp = jnp.where(m_xp, pltpu.roll(v, L - 1, axis=1), zero)
            c3 = jnp.concatenate([vxm, v, vxp], axis=0)

            def wgroup(g):
                return jnp.concatenate(
                    [w_ref[3 * g], w_ref[3 * g + 1], w_ref[3 * g + 2]],
                    axis=1).astype(jnp.bfloat16)

            g_m1 = jnp.dot(wgroup(0), c3, preferred_element_type=jnp.float32)
            g_0 = jnp.dot(wgroup(1), c3, preferred_element_type=jnp.float32)
            g_p1 = jnp.dot(wgroup(2), c3, preferred_element_type=jnp.float32)
            zf = jnp.zeros((), jnp.float32)
            g = g_0 + jnp.where(m_ym, pltpu.roll(g_m1, W, axis=1), zf)
            g = g + jnp.where(m_yp, pltpu.roll(g_p1, L - W, axis=1), zf)
            return jnp.maximum(g + bias, 0.0)

        h1 = conv3x3(h.astype(jnp.bfloat16), w1_ref, bvec_ref[1])
        h2 = conv3x3(h1.astype(jnp.bfloat16), w2_ref, bvec_ref[2])

        h_sc[i] = h2.astype(jnp.bfloat16)

        sel = (jax.lax.broadcasted_iota(jnp.int32, (P, L), 1) // HW ==
               jax.lax.broadcasted_iota(jnp.int32, (P, L), 0))
        selw = jnp.where(sel, 1.0 / HW, 0.0)
        pooled_sc[i] = jax.lax.dot_general(
            selw, h2, (((1,), (1,)), ((), ())),
            preferred_element_type=jnp.float32)

    @pl.when(i == M)
    def _gat():
        E = ei_ref.shape[1]
        neg_slope = 0.2
        ii = jax.lax.broadcasted_iota(jnp.int32, (N, E), 0)
        don = (ii == ei_ref[1:2, :]).astype(jnp.float32)
        son = (ii == ei_ref[0:1, :]).astype(jnp.float32)
        cnt = jax.lax.dot_general(don, son, (((1,), (1,)), ((), ())),
                                  preferred_element_type=jnp.float32)
        ri = jax.lax.broadcasted_iota(jnp.int32, (N, N), 0)
        ci = jax.lax.broadcasted_iota(jnp.int32, (N, N), 1)
        adj = jnp.logical_or(cnt > 0, ri == ci)

        xg = pooled_sc[...].reshape(N, hid)

        for l in range(num_layers):
            h_all = jnp.dot(xg, wgat_ref[l], preferred_element_type=jnp.float32)
            s_all = jax.lax.dot_general(usrc_ref[l], xg,
                                        (((0,), (1,)), ((), ())),
                                        preferred_element_type=jnp.float32)
            d_all = jnp.dot(xg, udst_ref[l], preferred_element_type=jnp.float32)
            acc = jnp.zeros((N, hid), jnp.float32)
            for hd in range(heads):
                e = d_all[:, hd:hd + 1] + s_all[hd:hd + 1, :]
                e = jnp.where(e > 0, e, neg_slope * e)
                e = jnp.where(adj, e, -1e9)
                e = e - jnp.max(e, axis=-1, keepdims=True)
                pr = jnp.exp(e)
                pr = pr / jnp.sum(pr, axis=-1, keepdims=True)
                acc = acc + jnp.dot(pr, h_all[:, hd * hid:(hd + 1) * hid],
                                    preferred_element_type=jnp.float32)
            xg = jnp.maximum(acc * (1.0 / heads) + bgat_ref[l], 0.0)

        zt = jax.lax.dot_general(xg, wout_ref[...], (((1,), (1,)), ((), ())),
                                 preferred_element_type=jnp.float32)
        c_sc[...] = zt.reshape(M, P, C)

    @pl.when(i > M)
    def _combine():
        j = i - M - 1
        wout_bf = wout_ref[...].astype(jnp.bfloat16)
        sel = (jax.lax.broadcasted_iota(jnp.int32, (P, L), 1) // HW ==
               jax.lax.broadcasted_iota(jnp.int32, (P, L), 0))
        selw = jnp.where(sel, 1.0, 0.0)
        for k in range(SB):
            y = jnp.dot(wout_bf, h_sc[j * SB + k],
                        preferred_element_type=jnp.float32)
            corr = jax.lax.dot_general(c_sc[j * SB + k], selw,
                                       (((0,), (0,)), ((), ())),
                                       preferred_element_type=jnp.float32)
            out_ref[k] = y + corr + bout_ref[...]


def kernel(x, edge_index, confidence_maps, w_enc, bvec, w_sp1, w_sp2,
           w_gat, u_src, u_dst, b_gat, w_out, b_out):
    N, C, H, W = x.shape
    HW = H * W
    hid = w_enc.shape[0]
    num_layers = w_gat.shape[0]
    heads = u_src.shape[2]
    P = 4 if N % 4 == 0 else 1
    M = N // P
    SB = 2 if M % 2 == 0 else 1
    L = P * HW
    G = M + 1 + M // SB

    x_flat = jnp.transpose(x.reshape(M, P, C, HW), (0, 2, 1, 3)) \
        .reshape(M, C, L).astype(jnp.bfloat16)
    conf_flat = jnp.transpose(confidence_maps.reshape(M, P, 1, HW),
                              (0, 2, 1, 3)).reshape(M, 1, L)

    body = functools.partial(_mega_body, H=H, W=W, P=P, SB=SB, M=M,
                             num_layers=num_layers, heads=heads)
    E = edge_index.shape[1]
    last_a = M - 1
    nblk = M // SB - 1

    out = pl.pallas_call(
        body,
        out_shape=jax.ShapeDtypeStruct((M, C, L), jnp.float32),
        grid=(G,),
        in_specs=[
            pl.BlockSpec((1, C, L), lambda i: (jnp.minimum(i, last_a), 0, 0)),
            pl.BlockSpec((1, 1, L), lambda i: (jnp.minimum(i, last_a), 0, 0)),
            pl.BlockSpec((2, E), lambda i: (0, 0)),
            pl.BlockSpec((hid, C), lambda i: (0, 0)),
            pl.BlockSpec((3, hid, 1), lambda i: (0, 0, 0)),
            pl.BlockSpec((9, hid, hid), lambda i: (0, 0, 0)),
            pl.BlockSpec((9, hid, hid), lambda i: (0, 0, 0)),
            pl.BlockSpec((num_layers, hid, heads * hid), lambda i: (0, 0, 0)),
            pl.BlockSpec((num_layers, hid, heads), lambda i: (0, 0, 0)),
            pl.BlockSpec((num_layers, hid, heads), lambda i: (0, 0, 0)),
            pl.BlockSpec((num_layers, 1, hid), lambda i: (0, 0, 0)),
            pl.BlockSpec((C, hid), lambda i: (0, 0)),
            pl.BlockSpec((C, 1), lambda i: (0, 0)),
        ],
        out_specs=pl.BlockSpec(
            (SB, C, L),
            lambda i: (jnp.clip(i - (M + 1), 0, nblk), 0, 0)),
        scratch_shapes=[
            pltpu.VMEM((M, hid, L), jnp.bfloat16),
            pltpu.VMEM((M, P, hid), jnp.float32),
            pltpu.VMEM((M, P, C), jnp.float32),
        ],
        compiler_params=pltpu.CompilerParams(
            dimension_semantics=("arbitrary",)),
    )(x_flat, conf_flat, edge_index, w_enc, bvec, w_sp1, w_sp2,
      w_gat, u_src, u_dst, b_gat, w_out, b_out)

    return jnp.transpose(out.reshape(M, C, P, H, W), (0, 2, 1, 3, 4)) \
        .reshape(N, C, H, W)


# speedup vs baseline: 1.0747x; 1.0330x over previous
import functools

import jax
import jax.numpy as jnp
from jax.experimental import pallas as pl
from jax.experimental.pallas import tpu as pltpu


def _mega_body(x_ref, conf_ref, ei_ref, wenc_ref, bvec_ref, w1_ref, w2_ref,
               wgat_ref, usrc_ref, udst_ref, bgat_ref, wout_ref, bout_ref,
               out_ref, h_sc, pooled_sc, c_sc,
               *, H, W, P, SB, M, num_layers, heads):
    HW = H * W
    L = P * HW
    hid = wenc_ref.shape[0]
    C = wout_ref.shape[0]
    N = M * P
    i = pl.program_id(0)

    @pl.when(i < M)
    def _spatial():
        x = x_ref[0]
        conf = conf_ref[0]

        h = jnp.dot(wenc_ref[...].astype(jnp.bfloat16), x,
                    preferred_element_type=jnp.float32)
        h = jnp.maximum(h + bvec_ref[0], 0.0) * conf

        p = jax.lax.broadcasted_iota(jnp.int32, (1, L), 1)
        xcol = p % W
        yrow = (p // W) % H
        m_xm = xcol >= 1
        m_xp = xcol <= W - 2
        m_ym = yrow >= 1
        m_yp = yrow <= H - 2

        def conv3x3(v, w_ref, bias):
            zero = jnp.zeros((), jnp.bfloat16)
            vxm = jnp.where(m_xm, pltpu.roll(v, 1, axis=1), zero)
            vxp = jnp.where(m_xp, pltpu.roll(v, L - 1, axis=1), zero)
            c3 = jnp.concatenate([vxm, v, vxp], axis=0)

            def wgroup(g):
                return jnp.concatenate(
                    [w_ref[3 * g], w_ref[3 * g + 1], w_ref[3 * g + 2]],
                    axis=1).astype(jnp.bfloat16)

            g_m1 = jnp.dot(wgroup(0), c3, preferred_element_type=jnp.float32)
            g_0 = jnp.dot(wgroup(1), c3, preferred_element_type=jnp.float32)
            g_p1 = jnp.dot(wgroup(2), c3, preferred_element_type=jnp.float32)
            zf = jnp.zeros((), jnp.float32)
            g = g_0 + jnp.where(m_ym, pltpu.roll(g_m1, W, axis=1), zf)
            g = g + jnp.where(m_yp, pltpu.roll(g_p1, L - W, axis=1), zf)
            return jnp.maximum(g + bias, 0.0)

        h1 = conv3x3(h.astype(jnp.bfloat16), w1_ref, bvec_ref[1])
        h2 = conv3x3(h1.astype(jnp.bfloat16), w2_ref, bvec_ref[2])

        h_sc[i] = h2.astype(jnp.bfloat16)

        sel = (jax.lax.broadcasted_iota(jnp.int32, (P, L), 1) // HW ==
               jax.lax.broadcasted_iota(jnp.int32, (P, L), 0))
        selw = jnp.where(sel, 1.0 / HW, 0.0)
        pooled_sc[i] = jax.lax.dot_general(
            selw, h2, (((1,), (1,)), ((), ())),
            preferred_element_type=jnp.float32)

    @pl.when(i == M)
    def _gat():
        E = ei_ref.shape[1]
        neg_slope = 0.2
        ii = jax.lax.broadcasted_iota(jnp.int32, (N, E), 0).astype(jnp.float32)
        don = (ii == ei_ref[1:2, :]).astype(jnp.float32)
        son = (ii == ei_ref[0:1, :]).astype(jnp.float32)
        cnt = jax.lax.dot_general(don, son, (((1,), (1,)), ((), ())),
                                  preferred_element_type=jnp.float32)
        ri = jax.lax.broadcasted_iota(jnp.int32, (N, N), 0)
        ci = jax.lax.broadcasted_iota(jnp.int32, (N, N), 1)
        adj = jnp.logical_or(cnt > 0, ri == ci)

        xg = pooled_sc[...].reshape(N, hid)

        for l in range(num_layers):
            h_all = jnp.dot(xg, wgat_ref[l], preferred_element_type=jnp.float32)
            s_all = jax.lax.dot_general(usrc_ref[l], xg,
                                        (((0,), (1,)), ((), ())),
                                        preferred_element_type=jnp.float32)
            d_all = jnp.dot(xg, udst_ref[l], preferred_element_type=jnp.float32)
            acc = jnp.zeros((N, hid), jnp.float32)
            for hd in range(heads):
                e = d_all[:, hd:hd + 1] + s_all[hd:hd + 1, :]
                e = jnp.where(e > 0, e, neg_slope * e)
                e = jnp.where(adj, e, -1e9)
                e = e - jnp.max(e, axis=-1, keepdims=True)
                pr = jnp.exp(e)
                pr = pr / jnp.sum(pr, axis=-1, keepdims=True)
                acc = acc + jnp.dot(pr, h_all[:, hd * hid:(hd + 1) * hid],
                                    preferred_element_type=jnp.float32)
            xg = jnp.maximum(acc * (1.0 / heads) + bgat_ref[l], 0.0)

        zt = jax.lax.dot_general(xg, wout_ref[...], (((1,), (1,)), ((), ())),
                                 preferred_element_type=jnp.float32)
        c_sc[...] = zt.reshape(M, P, C)

    @pl.when(i > M)
    def _combine():
        j = i - M - 1
        wout_bf = wout_ref[...].astype(jnp.bfloat16)
        sel = (jax.lax.broadcasted_iota(jnp.int32, (P, L), 1) // HW ==
               jax.lax.broadcasted_iota(jnp.int32, (P, L), 0))
        selw = jnp.where(sel, 1.0, 0.0)
        for k in range(SB):
            y = jnp.dot(wout_bf, h_sc[j * SB + k],
                        preferred_element_type=jnp.float32)
            corr = jax.lax.dot_general(c_sc[j * SB + k], selw,
                                       (((0,), (0,)), ((), ())),
                                       preferred_element_type=jnp.float32)
            out_ref[k] = y + corr + bout_ref[...]


def kernel(x, edge_index, confidence_maps, w_enc, bvec, w_sp1, w_sp2,
           w_gat, u_src, u_dst, b_gat, w_out, b_out):
    N, C, H, W = x.shape
    HW = H * W
    hid = w_enc.shape[0]
    num_layers = w_gat.shape[0]
    heads = u_src.shape[2]
    P = 8 if N % 8 == 0 else 1
    M = N // P
    SB = 2 if M % 2 == 0 else 1
    L = P * HW
    G = M + 1 + M // SB

    x_flat = jnp.transpose(x.reshape(M, P, C, HW), (0, 2, 1, 3)) \
        .reshape(M, C, L).astype(jnp.bfloat16)
    conf_flat = jnp.transpose(confidence_maps.reshape(M, P, 1, HW),
                              (0, 2, 1, 3)).reshape(M, 1, L)

    body = functools.partial(_mega_body, H=H, W=W, P=P, SB=SB, M=M,
                             num_layers=num_layers, heads=heads)
    E = edge_index.shape[1]
    last_a = M - 1
    nblk = M // SB - 1

    out = pl.pallas_call(
        body,
        out_shape=jax.ShapeDtypeStruct((M, C, L), jnp.float32),
        grid=(G,),
        in_specs=[
            pl.BlockSpec((1, C, L), lambda i: (jnp.minimum(i, last_a), 0, 0)),
            pl.BlockSpec((1, 1, L), lambda i: (jnp.minimum(i, last_a), 0, 0)),
            pl.BlockSpec((2, E), lambda i: (0, 0)),
            pl.BlockSpec((hid, C), lambda i: (0, 0)),
            pl.BlockSpec((3, hid, 1), lambda i: (0, 0, 0)),
            pl.BlockSpec((9, hid, hid), lambda i: (0, 0, 0)),
            pl.BlockSpec((9, hid, hid), lambda i: (0, 0, 0)),
            pl.BlockSpec((num_layers, hid, heads * hid), lambda i: (0, 0, 0)),
            pl.BlockSpec((num_layers, hid, heads), lambda i: (0, 0, 0)),
            pl.BlockSpec((num_layers, hid, heads), lambda i: (0, 0, 0)),
            pl.BlockSpec((num_layers, 1, hid), lambda i: (0, 0, 0)),
            pl.BlockSpec((C, hid), lambda i: (0, 0)),
            pl.BlockSpec((C, 1), lambda i: (0, 0)),
        ],
        out_specs=pl.BlockSpec(
            (SB, C, L),
            lambda i: (jnp.clip(i - (M + 1), 0, nblk), 0, 0)),
        scratch_shapes=[
            pltpu.VMEM((M, hid, L), jnp.bfloat16),
            pltpu.VMEM((M, P, hid), jnp.float32),
            pltpu.VMEM((M, P, C), jnp.float32),
        ],
        compiler_params=pltpu.CompilerParams(
            dimension_semantics=("arbitrary",)),
    )(x_flat, conf_flat, edge_index.astype(jnp.float32), w_enc, bvec,
      w_sp1, w_sp2, w_gat, u_src, u_dst, b_gat, w_out, b_out)

    return jnp.transpose(out.reshape(M, C, P, H, W), (0, 2, 1, 3, 4)) \
        .reshape(N, C, H, W)


# speedup vs baseline: 1.0766x; 1.0017x over previous
import functools

import jax
import jax.numpy as jnp
from jax.experimental import pallas as pl
from jax.experimental.pallas import tpu as pltpu


def _mega_body(x_ref, conf_ref, ei_ref, wenc_ref, bvec_ref, w1_ref, w2_ref,
               wgat_ref, usrc_ref, udst_ref, bgat_ref, wout_ref, bout_ref,
               out_ref, h_sc, pooled_sc, c_sc,
               *, H, W, P, SB, M, num_layers, heads):
    HW = H * W
    L = P * HW
    hid = wenc_ref.shape[0]
    C = wout_ref.shape[0]
    N = M * P
    i = pl.program_id(0)

    @pl.when(i < M)
    def _spatial():
        x = x_ref[0]
        conf = conf_ref[0]

        h = jnp.dot(wenc_ref[...].astype(jnp.bfloat16), x,
                    preferred_element_type=jnp.float32)
        h = jnp.maximum(h + bvec_ref[0], 0.0) * conf

        p = jax.lax.broadcasted_iota(jnp.int32, (1, L), 1)
        xcol = p % W
        yrow = (p // W) % H
        m_xm = xcol >= 1
        m_xp = xcol <= W - 2
        m_ym = yrow >= 1
        m_yp = yrow <= H - 2

        def conv3x3(v, w_ref, bias):
            zero = jnp.zeros((), jnp.bfloat16)
            vxm = jnp.where(m_xm, pltpu.roll(v, 1, axis=1), zero)
            vxp = jnp.where(m_xp, pltpu.roll(v, L - 1, axis=1), zero)
            c3 = jnp.concatenate([vxm, v, vxp], axis=0)

            def wgroup(g):
                return jnp.concatenate(
                    [w_ref[3 * g], w_ref[3 * g + 1], w_ref[3 * g + 2]],
                    axis=1).astype(jnp.bfloat16)

            g_m1 = jnp.dot(wgroup(0), c3, preferred_element_type=jnp.float32)
            g_0 = jnp.dot(wgroup(1), c3, preferred_element_type=jnp.float32)
            g_p1 = jnp.dot(wgroup(2), c3, preferred_element_type=jnp.float32)
            zf = jnp.zeros((), jnp.float32)
            g = g_0 + jnp.where(m_ym, pltpu.roll(g_m1, W, axis=1), zf)
            g = g + jnp.where(m_yp, pltpu.roll(g_p1, L - W, axis=1), zf)
            return jnp.maximum(g + bias, 0.0)

        h1 = conv3x3(h.astype(jnp.bfloat16), w1_ref, bvec_ref[1])
        h2 = conv3x3(h1.astype(jnp.bfloat16), w2_ref, bvec_ref[2])

        h_sc[i] = h2.astype(jnp.bfloat16)

        sel = (jax.lax.broadcasted_iota(jnp.int32, (P, L), 1) // HW ==
               jax.lax.broadcasted_iota(jnp.int32, (P, L), 0))
        selw = jnp.where(sel, 1.0 / HW, 0.0)
        pooled_sc[i] = jax.lax.dot_general(
            selw, h2, (((1,), (1,)), ((), ())),
            preferred_element_type=jnp.float32)

    @pl.when(i == M)
    def _gat():
        E = ei_ref.shape[1]
        neg_slope = 0.2
        ii = jax.lax.broadcasted_iota(jnp.int32, (N, E), 0).astype(jnp.float32)
        don = (ii == ei_ref[1:2, :]).astype(jnp.float32)
        son = (ii == ei_ref[0:1, :]).astype(jnp.float32)
        cnt = jax.lax.dot_general(don, son, (((1,), (1,)), ((), ())),
                                  preferred_element_type=jnp.float32)
        ri = jax.lax.broadcasted_iota(jnp.int32, (N, N), 0)
        ci = jax.lax.broadcasted_iota(jnp.int32, (N, N), 1)
        adj = jnp.logical_or(cnt > 0, ri == ci)

        xg = pooled_sc[...].reshape(N, hid)

        for l in range(num_layers):
            h_all = jnp.dot(xg, wgat_ref[l], preferred_element_type=jnp.float32)
            s_all = jax.lax.dot_general(usrc_ref[l], xg,
                                        (((1,), (1,)), ((), ())),
                                        preferred_element_type=jnp.float32)
            d_all = jax.lax.dot_general(xg, udst_ref[l],
                                        (((1,), (1,)), ((), ())),
                                        preferred_element_type=jnp.float32)
            acc = jnp.zeros((N, hid), jnp.float32)
            for hd in range(heads):
                e = d_all[:, hd:hd + 1] + s_all[hd:hd + 1, :]
                e = jnp.where(e > 0, e, neg_slope * e)
                e = jnp.where(adj, e, -1e9)
                e = e - jnp.max(e, axis=-1, keepdims=True)
                pr = jnp.exp(e)
                pr = pr / jnp.sum(pr, axis=-1, keepdims=True)
                acc = acc + jnp.dot(pr, h_all[:, hd * hid:(hd + 1) * hid],
                                    preferred_element_type=jnp.float32)
            xg = jnp.maximum(acc * (1.0 / heads) + bgat_ref[l], 0.0)

        zt = jax.lax.dot_general(xg, wout_ref[...], (((1,), (1,)), ((), ())),
                                 preferred_element_type=jnp.float32)
        c_sc[...] = zt.reshape(M, P, C)

    @pl.when(i > M)
    def _combine():
        j = i - M - 1
        wout_bf = wout_ref[...].astype(jnp.bfloat16)
        sel = (jax.lax.broadcasted_iota(jnp.int32, (P, L), 1) // HW ==
               jax.lax.broadcasted_iota(jnp.int32, (P, L), 0))
        selw = jnp.where(sel, 1.0, 0.0)
        for k in range(SB):
            y = jnp.dot(wout_bf, h_sc[j * SB + k],
                        preferred_element_type=jnp.float32)
            corr = jax.lax.dot_general(c_sc[j * SB + k], selw,
                                       (((0,), (0,)), ((), ())),
                                       preferred_element_type=jnp.float32)
            out_ref[k] = y + corr + bout_ref[...]


def kernel(x, edge_index, confidence_maps, w_enc, bvec, w_sp1, w_sp2,
           w_gat, u_src, u_dst, b_gat, w_out, b_out):
    N, C, H, W = x.shape
    HW = H * W
    hid = w_enc.shape[0]
    num_layers = w_gat.shape[0]
    heads = u_src.shape[2]
    P = 8 if N % 8 == 0 else 1
    M = N // P
    SB = 2 if M % 2 == 0 else 1
    L = P * HW
    G = M + 1 + M // SB

    x_flat = jnp.transpose(x.reshape(M, P, C, HW), (0, 2, 1, 3)) \
        .reshape(M, C, L).astype(jnp.bfloat16)
    conf_flat = jnp.transpose(confidence_maps.reshape(M, P, 1, HW),
                              (0, 2, 1, 3)).reshape(M, 1, L)

    body = functools.partial(_mega_body, H=H, W=W, P=P, SB=SB, M=M,
                             num_layers=num_layers, heads=heads)
    E = edge_index.shape[1]
    last_a = M - 1
    nblk = M // SB - 1

    out = pl.pallas_call(
        body,
        out_shape=jax.ShapeDtypeStruct((M, C, L), jnp.float32),
        grid=(G,),
        in_specs=[
            pl.BlockSpec((1, C, L), lambda i: (jnp.minimum(i, last_a), 0, 0)),
            pl.BlockSpec((1, 1, L), lambda i: (jnp.minimum(i, last_a), 0, 0)),
            pl.BlockSpec((2, E), lambda i: (0, 0)),
            pl.BlockSpec((hid, C), lambda i: (0, 0)),
            pl.BlockSpec((3, hid, 1), lambda i: (0, 0, 0)),
            pl.BlockSpec((9, hid, hid), lambda i: (0, 0, 0)),
            pl.BlockSpec((9, hid, hid), lambda i: (0, 0, 0)),
            pl.BlockSpec((num_layers, hid, heads * hid), lambda i: (0, 0, 0)),
            pl.BlockSpec((num_layers, heads, hid), lambda i: (0, 0, 0)),
            pl.BlockSpec((num_layers, heads, hid), lambda i: (0, 0, 0)),
            pl.BlockSpec((num_layers, 1, hid), lambda i: (0, 0, 0)),
            pl.BlockSpec((C, hid), lambda i: (0, 0)),
            pl.BlockSpec((C, 1), lambda i: (0, 0)),
        ],
        out_specs=pl.BlockSpec(
            (SB, C, L),
            lambda i: (jnp.clip(i - (M + 1), 0, nblk), 0, 0)),
        scratch_shapes=[
            pltpu.VMEM((M, hid, L), jnp.bfloat16),
            pltpu.VMEM((M, P, hid), jnp.float32),
            pltpu.VMEM((M, P, C), jnp.float32),
        ],
        compiler_params=pltpu.CompilerParams(
            dimension_semantics=("arbitrary",)),
    )(x_flat, conf_flat, edge_index.astype(jnp.float32), w_enc, bvec,
      w_sp1, w_sp2, w_gat, jnp.transpose(u_src, (0, 2, 1)),
      jnp.transpose(u_dst, (0, 2, 1)), b_gat, w_out, b_out)

    return jnp.transpose(out.reshape(M, C, P, H, W), (0, 2, 1, 3, 4)) \
        .reshape(N, C, H, W)
